# baseline (speedup 1.0000x reference)
import functools

import jax
import jax.numpy as jnp
from jax import lax
from jax.experimental import pallas as pl
from jax.experimental.pallas import tpu as pltpu
from jax.experimental.pallas import tpu_sc as plsc

DIM = 128
NC = 2
NS = 16
NW = NC * NS
CHUNK = 128


K0, K1 = 136, 24
G0, G1 = 8, 8
STAGES = 5


def _sc_segment_sum(x, src_t, dst_t, n_acc, u):
    rows_pt = n_acc // NS
    k0, k1, g0, g1 = K0 * u, K1 * u, G0, G1
    s0, s1 = k0 // g0, k1 // g1

    mesh = plsc.VectorSubcoreMesh(core_axis_name="c", subcore_axis_name="s")

    @functools.partial(
        pl.kernel,
        out_type=jax.ShapeDtypeStruct((NC, n_acc, DIM), jnp.float32),
        mesh=mesh,
        scratch_types=[
            pltpu.VMEM((G0, CHUNK), jnp.int32),
            pltpu.VMEM((G0, CHUNK), jnp.int32),
            pltpu.VMEM((G0, CHUNK), jnp.int32),
            pltpu.VMEM((G0, CHUNK), jnp.int32),
            pltpu.VMEM((CHUNK, DIM), jnp.float32),
            pltpu.VMEM((CHUNK, DIM), jnp.float32),
            pltpu.VMEM_SHARED((n_acc, DIM), jnp.float32),
            pltpu.SemaphoreType.DMA,
            pltpu.SemaphoreType.DMA,
            pltpu.SemaphoreType.DMA,
            pltpu.SemaphoreType.DMA,
        ],
    )
    def kern(x_hbm, src_hbm, dst_hbm, out_hbm,
             src_v0, dst_v0, src_v1, dst_v1, rows_a, rows_b, acc,
             sem_a, sem_b, isem0, isem1):
        cid = lax.axis_index("c")
        sid = lax.axis_index("s")
        srcs, dsts, isems = (src_v0, src_v1), (dst_v0, dst_v1), (isem0, isem1)

        def zrow(i, carry):
            for l in range(DIM // 16):
                rows_a[i, pl.ds(l * 16, 16)] = jnp.zeros((16,), jnp.float32)
            return carry

        lax.fori_loop(0, CHUNK, zrow, 0)
        base = sid * rows_pt
        full, rem = rows_pt // CHUNK, rows_pt % CHUNK
        for b in range(full):
            pltpu.sync_copy(rows_a, acc.at[pl.ds(base + b * CHUNK, CHUNK)])
        if rem:
            pltpu.sync_copy(rows_a.at[pl.ds(0, rem)],
                            acc.at[pl.ds(base + full * CHUNK, rem)])
        plsc.subcore_barrier()

        def pipeline(start_c, g, stages):
            pltpu.async_copy(src_hbm.at[pl.ds(start_c, g)],
                             srcs[0].at[pl.ds(0, g)], isems[0])
            pltpu.async_copy(dst_hbm.at[pl.ds(start_c, g)],
                             dsts[0].at[pl.ds(0, g)], isems[0])
            for st in range(stages):
                cur, nxt = st % 2, (st + 1) % 2
                sv, dv = srcs[cur], dsts[cur]
                pltpu.make_async_copy(
                    src_hbm.at[pl.ds(start_c, g)],
                    sv.at[pl.ds(0, g)], isems[cur]).wait()
                pltpu.make_async_copy(
                    dst_hbm.at[pl.ds(start_c, g)],
                    dv.at[pl.ds(0, g)], isems[cur]).wait()
                if st + 1 < stages:
                    c1 = start_c + (st + 1) * g
                    pltpu.async_copy(src_hbm.at[pl.ds(c1, g)],
                                     srcs[nxt].at[pl.ds(0, g)], isems[nxt])
                    pltpu.async_copy(dst_hbm.at[pl.ds(c1, g)],
                                     dsts[nxt].at[pl.ds(0, g)], isems[nxt])
                pltpu.async_copy(x_hbm.at[sv.at[0]], rows_a, sem_a)

                def step(i, carry, sv=sv, dv=dv, g=g):
                    j0 = 2 * i
                    j1 = j0 + 1
                    pltpu.make_async_copy(
                        x_hbm.at[sv.at[0]], rows_a, sem_a).wait()
                    pltpu.async_copy(x_hbm.at[sv.at[j1]], rows_b, sem_b)
                    pltpu.sync_copy(rows_a, acc.at[dv.at[j0]], add=True)
                    pltpu.make_async_copy(
                        x_hbm.at[sv.at[0]], rows_b, sem_b).wait()

                    @pl.when(j1 + 1 < g)
                    def _():
                        pltpu.async_copy(
                            x_hbm.at[sv.at[j1 + 1]], rows_a, sem_a)

                    pltpu.sync_copy(rows_b, acc.at[dv.at[j1]], add=True)
                    return carry

                lax.fori_loop(0, g // 2, step, 0)

        @pl.when(cid == 0)
        def _():
            pipeline(sid * k0, g0, s0)

        @pl.when(cid == 1)
        def _():
            pipeline(NS * k0 + sid * k1, g1, s1)

        plsc.subcore_barrier()
        pltpu.sync_copy(acc.at[pl.ds(base, rows_pt)],
                        out_hbm.at[cid, pl.ds(base, rows_pt)])

    return kern(x, src_t, dst_t)


def _gru_body(parts_ref, x_ref, wih_ref, whh_ref, bih_ref, bhh_ref,
              gam_ref, bet_ref, o_ref):
    msg = parts_ref[0] + parts_ref[1]
    xb = x_ref[...]
    dn = (((1,), (1,)), ((), ()))
    gi = lax.dot_general(msg, wih_ref[...], dn,
                         preferred_element_type=jnp.float32) + bih_ref[...]
    gh = lax.dot_general(xb, whh_ref[...], dn,
                         preferred_element_type=jnp.float32) + bhh_ref[...]
    r = jax.nn.sigmoid(gi[:, :DIM] + gh[:, :DIM])
    z = jax.nn.sigmoid(gi[:, DIM:2 * DIM] + gh[:, DIM:2 * DIM])
    n = jnp.tanh(gi[:, 2 * DIM:] + r * gh[:, 2 * DIM:])
    h = (1.0 - z) * n + z * xb
    mu = jnp.mean(h, axis=1, keepdims=True)
    d = h - mu
    var = jnp.mean(d * d, axis=1, keepdims=True)
    o_ref[...] = d * lax.rsqrt(var + 1e-5) * gam_ref[...] + bet_ref[...]


def _gru_tc(parts, x, w_ih, w_hh, b_ih, b_hh, gamma, beta, blk):
    n = x.shape[0]
    grid = (n // blk,)
    return pl.pallas_call(
        _gru_body,
        grid=grid,
        in_specs=[
            pl.BlockSpec((NC, blk, DIM), lambda i: (0, i, 0)),
            pl.BlockSpec((blk, DIM), lambda i: (i, 0)),
            pl.BlockSpec((3 * DIM, DIM), lambda i: (0, 0)),
            pl.BlockSpec((3 * DIM, DIM), lambda i: (0, 0)),
            pl.BlockSpec((1, 3 * DIM), lambda i: (0, 0)),
            pl.BlockSpec((1, 3 * DIM), lambda i: (0, 0)),
            pl.BlockSpec((1, DIM), lambda i: (0, 0)),
            pl.BlockSpec((1, DIM), lambda i: (0, 0)),
        ],
        out_specs=pl.BlockSpec((blk, DIM), lambda i: (i, 0)),
        out_shape=jax.ShapeDtypeStruct((n, DIM), jnp.float32),
    )(parts, x, w_ih, w_hh, b_ih, b_hh, gamma, beta)


@jax.jit
def kernel(x, edge_index, W_ih, W_hh, b_ih, b_hh, ln_gamma, ln_beta):
    n = x.shape[0]
    e = edge_index.shape[1]

    per_u = CHUNK * NS * (K0 + K1)
    u = -(-e // per_u)
    e_pad = u * per_u
    src = jnp.concatenate(
        [edge_index[0], jnp.zeros((e_pad - e,), jnp.int32)]).reshape(-1, CHUNK)
    dst = jnp.concatenate(
        [edge_index[1], jnp.full((e_pad - e,), n, jnp.int32)]).reshape(-1, CHUNK)

    n_acc = -(-(n + 1) // (NS * 8)) * (NS * 8)

    parts = _sc_segment_sum(x, src, dst, n_acc, u)
    parts = lax.slice(parts, (0, 0, 0), (NC, n, DIM))

    return _gru_tc(parts, x, W_ih, W_hh, b_ih.reshape(1, -1),
                   b_hh.reshape(1, -1), ln_gamma.reshape(1, -1),
                   ln_beta.reshape(1, -1), blk=1000)


# speedup vs baseline: 1.0811x; 1.0811x over previous
import functools

import jax
import jax.numpy as jnp
from jax import lax
from jax.experimental import pallas as pl
from jax.experimental.pallas import tpu as pltpu
from jax.experimental.pallas import tpu_sc as plsc

DIM = 128
NC = 2
NS = 16
NW = NC * NS
CHUNK = 128


K0, K1 = 144, 16
G0, G1 = 24, 16
STAGES = 5


def _sc_segment_sum(x, src_t, dst_t, n_acc, u):
    rows_pt = n_acc // NS
    k0, k1, g0, g1 = K0 * u, K1 * u, G0, G1
    s0, s1 = k0 // g0, k1 // g1

    mesh = plsc.VectorSubcoreMesh(core_axis_name="c", subcore_axis_name="s")

    @functools.partial(
        pl.kernel,
        out_type=jax.ShapeDtypeStruct((NC, n_acc, DIM), jnp.float32),
        mesh=mesh,
        scratch_types=[
            pltpu.VMEM((G0, CHUNK), jnp.int32),
            pltpu.VMEM((G0, CHUNK), jnp.int32),
            pltpu.VMEM((G0, CHUNK), jnp.int32),
            pltpu.VMEM((G0, CHUNK), jnp.int32),
            pltpu.VMEM((CHUNK, DIM), jnp.float32),
            pltpu.VMEM((CHUNK, DIM), jnp.float32),
            pltpu.VMEM_SHARED((n_acc, DIM), jnp.float32),
            pltpu.SemaphoreType.DMA,
            pltpu.SemaphoreType.DMA,
            pltpu.SemaphoreType.DMA,
            pltpu.SemaphoreType.DMA,
        ],
    )
    def kern(x_hbm, src_hbm, dst_hbm, out_hbm,
             src_v0, dst_v0, src_v1, dst_v1, rows_a, rows_b, acc,
             sem_a, sem_b, isem0, isem1):
        cid = lax.axis_index("c")
        sid = lax.axis_index("s")
        srcs, dsts, isems = (src_v0, src_v1), (dst_v0, dst_v1), (isem0, isem1)

        def zrow(i, carry):
            for l in range(DIM // 16):
                rows_a[i, pl.ds(l * 16, 16)] = jnp.zeros((16,), jnp.float32)
            return carry

        lax.fori_loop(0, CHUNK, zrow, 0)
        base = sid * rows_pt
        full, rem = rows_pt // CHUNK, rows_pt % CHUNK
        for b in range(full):
            pltpu.sync_copy(rows_a, acc.at[pl.ds(base + b * CHUNK, CHUNK)])
        if rem:
            pltpu.sync_copy(rows_a.at[pl.ds(0, rem)],
                            acc.at[pl.ds(base + full * CHUNK, rem)])
        plsc.subcore_barrier()

        def pipeline(start_c, g, stages):
            pltpu.async_copy(src_hbm.at[pl.ds(start_c, g)],
                             srcs[0].at[pl.ds(0, g)], isems[0])
            pltpu.async_copy(dst_hbm.at[pl.ds(start_c, g)],
                             dsts[0].at[pl.ds(0, g)], isems[0])
            for st in range(stages):
                cur, nxt = st % 2, (st + 1) % 2
                sv, dv = srcs[cur], dsts[cur]
                pltpu.make_async_copy(
                    src_hbm.at[pl.ds(start_c, g)],
                    sv.at[pl.ds(0, g)], isems[cur]).wait()
                pltpu.make_async_copy(
                    dst_hbm.at[pl.ds(start_c, g)],
                    dv.at[pl.ds(0, g)], isems[cur]).wait()
                if st + 1 < stages:
                    c1 = start_c + (st + 1) * g
                    pltpu.async_copy(src_hbm.at[pl.ds(c1, g)],
                                     srcs[nxt].at[pl.ds(0, g)], isems[nxt])
                    pltpu.async_copy(dst_hbm.at[pl.ds(c1, g)],
                                     dsts[nxt].at[pl.ds(0, g)], isems[nxt])
                pltpu.async_copy(x_hbm.at[sv.at[0]], rows_a, sem_a)

                def step(i, carry, sv=sv, dv=dv, g=g):
                    j0 = 2 * i
                    j1 = j0 + 1
                    pltpu.make_async_copy(
                        x_hbm.at[sv.at[0]], rows_a, sem_a).wait()
                    pltpu.async_copy(x_hbm.at[sv.at[j1]], rows_b, sem_b)
                    pltpu.sync_copy(rows_a, acc.at[dv.at[j0]], add=True)
                    pltpu.make_async_copy(
                        x_hbm.at[sv.at[0]], rows_b, sem_b).wait()

                    @pl.when(j1 + 1 < g)
                    def _():
                        pltpu.async_copy(
                            x_hbm.at[sv.at[j1 + 1]], rows_a, sem_a)

                    pltpu.sync_copy(rows_b, acc.at[dv.at[j1]], add=True)
                    return carry

                lax.fori_loop(0, g // 2, step, 0)

        @pl.when(cid == 0)
        def _():
            pipeline(sid * k0, g0, s0)

        @pl.when(cid == 1)
        def _():
            pipeline(NS * k0 + sid * k1, g1, s1)

        plsc.subcore_barrier()
        pltpu.sync_copy(acc.at[pl.ds(base, rows_pt)],
                        out_hbm.at[cid, pl.ds(base, rows_pt)])

    return kern(x, src_t, dst_t)


def _gru_body(parts_ref, x_ref, wih_ref, whh_ref, bih_ref, bhh_ref,
              gam_ref, bet_ref, o_ref):
    msg = parts_ref[0] + parts_ref[1]
    xb = x_ref[...]
    dn = (((1,), (1,)), ((), ()))
    gi = lax.dot_general(msg, wih_ref[...], dn,
                         preferred_element_type=jnp.float32) + bih_ref[...]
    gh = lax.dot_general(xb, whh_ref[...], dn,
                         preferred_element_type=jnp.float32) + bhh_ref[...]
    r = jax.nn.sigmoid(gi[:, :DIM] + gh[:, :DIM])
    z = jax.nn.sigmoid(gi[:, DIM:2 * DIM] + gh[:, DIM:2 * DIM])
    n = jnp.tanh(gi[:, 2 * DIM:] + r * gh[:, 2 * DIM:])
    h = (1.0 - z) * n + z * xb
    mu = jnp.mean(h, axis=1, keepdims=True)
    d = h - mu
    var = jnp.mean(d * d, axis=1, keepdims=True)
    o_ref[...] = d * lax.rsqrt(var + 1e-5) * gam_ref[...] + bet_ref[...]


def _gru_tc(parts, x, w_ih, w_hh, b_ih, b_hh, gamma, beta, blk):
    n = x.shape[0]
    grid = (n // blk,)
    return pl.pallas_call(
        _gru_body,
        grid=grid,
        in_specs=[
            pl.BlockSpec((NC, blk, DIM), lambda i: (0, i, 0)),
            pl.BlockSpec((blk, DIM), lambda i: (i, 0)),
            pl.BlockSpec((3 * DIM, DIM), lambda i: (0, 0)),
            pl.BlockSpec((3 * DIM, DIM), lambda i: (0, 0)),
            pl.BlockSpec((1, 3 * DIM), lambda i: (0, 0)),
            pl.BlockSpec((1, 3 * DIM), lambda i: (0, 0)),
            pl.BlockSpec((1, DIM), lambda i: (0, 0)),
            pl.BlockSpec((1, DIM), lambda i: (0, 0)),
        ],
        out_specs=pl.BlockSpec((blk, DIM), lambda i: (i, 0)),
        out_shape=jax.ShapeDtypeStruct((n, DIM), jnp.float32),
    )(parts, x, w_ih, w_hh, b_ih, b_hh, gamma, beta)


@jax.jit
def kernel(x, edge_index, W_ih, W_hh, b_ih, b_hh, ln_gamma, ln_beta):
    n = x.shape[0]
    e = edge_index.shape[1]

    per_u = CHUNK * NS * (K0 + K1)
    u = -(-e // per_u)
    e_pad = u * per_u
    src = jnp.concatenate(
        [edge_index[0], jnp.zeros((e_pad - e,), jnp.int32)]).reshape(-1, CHUNK)
    dst = jnp.concatenate(
        [edge_index[1], jnp.full((e_pad - e,), n, jnp.int32)]).reshape(-1, CHUNK)

    n_acc = -(-(n + 1) // (NS * 8)) * (NS * 8)

    parts = _sc_segment_sum(x, src, dst, n_acc, u)
    parts = lax.slice(parts, (0, 0, 0), (NC, n, DIM))

    return _gru_tc(parts, x, W_ih, W_hh, b_ih.reshape(1, -1),
                   b_hh.reshape(1, -1), ln_gamma.reshape(1, -1),
                   ln_beta.reshape(1, -1), blk=1000)


# speedup vs baseline: 1.0879x; 1.0063x over previous
import functools

import jax
import jax.numpy as jnp
from jax import lax
from jax.experimental import pallas as pl
from jax.experimental.pallas import tpu as pltpu
from jax.experimental.pallas import tpu_sc as plsc

DIM = 128
NC = 2
NS = 16
NW = NC * NS
CHUNK = 128


K0, K1 = 144, 16
G0, G1 = 24, 16
STAGES = 5


def _sc_segment_sum(x, src_t, dst_t, n_acc, u):
    rows_pt = n_acc // NS
    k0, k1, g0, g1 = K0 * u, K1 * u, G0, G1
    s0, s1 = k0 // g0, k1 // g1

    mesh = plsc.VectorSubcoreMesh(core_axis_name="c", subcore_axis_name="s")

    @functools.partial(
        pl.kernel,
        out_type=jax.ShapeDtypeStruct((NC, n_acc, DIM), jnp.float32),
        mesh=mesh,
        scratch_types=[
            pltpu.VMEM((G0, CHUNK), jnp.int32),
            pltpu.VMEM((G0, CHUNK), jnp.int32),
            pltpu.VMEM((G0, CHUNK), jnp.int32),
            pltpu.VMEM((G0, CHUNK), jnp.int32),
            pltpu.VMEM((CHUNK, DIM), jnp.float32),
            pltpu.VMEM((CHUNK, DIM), jnp.float32),
            pltpu.VMEM_SHARED((n_acc, DIM), jnp.float32),
            pltpu.SemaphoreType.DMA,
            pltpu.SemaphoreType.DMA,
            pltpu.SemaphoreType.DMA,
            pltpu.SemaphoreType.DMA,
        ],
    )
    def kern(x_hbm, src_hbm, dst_hbm, out_hbm,
             src_v0, dst_v0, src_v1, dst_v1, rows_a, rows_b, acc,
             sem_a, sem_b, isem0, isem1):
        cid = lax.axis_index("c")
        sid = lax.axis_index("s")
        srcs, dsts, isems = (src_v0, src_v1), (dst_v0, dst_v1), (isem0, isem1)

        def zrow(i, carry):
            for l in range(DIM // 16):
                rows_a[i, pl.ds(l * 16, 16)] = jnp.zeros((16,), jnp.float32)
            return carry

        lax.fori_loop(0, CHUNK, zrow, 0)
        base = sid * rows_pt
        full, rem = rows_pt // CHUNK, rows_pt % CHUNK
        for b in range(full):
            pltpu.sync_copy(rows_a, acc.at[pl.ds(base + b * CHUNK, CHUNK)])
        if rem:
            pltpu.sync_copy(rows_a.at[pl.ds(0, rem)],
                            acc.at[pl.ds(base + full * CHUNK, rem)])
        plsc.subcore_barrier()

        def pipeline(start_c, g, stages):
            pltpu.async_copy(src_hbm.at[pl.ds(start_c, g)],
                             srcs[0].at[pl.ds(0, g)], isems[0])
            pltpu.async_copy(dst_hbm.at[pl.ds(start_c, g)],
                             dsts[0].at[pl.ds(0, g)], isems[0])
            for st in range(stages):
                cur, nxt = st % 2, (st + 1) % 2
                sv, dv = srcs[cur], dsts[cur]
                pltpu.make_async_copy(
                    src_hbm.at[pl.ds(start_c, g)],
                    sv.at[pl.ds(0, g)], isems[cur]).wait()
                pltpu.make_async_copy(
                    dst_hbm.at[pl.ds(start_c, g)],
                    dv.at[pl.ds(0, g)], isems[cur]).wait()
                if st + 1 < stages:
                    c1 = start_c + (st + 1) * g
                    pltpu.async_copy(src_hbm.at[pl.ds(c1, g)],
                                     srcs[nxt].at[pl.ds(0, g)], isems[nxt])
                    pltpu.async_copy(dst_hbm.at[pl.ds(c1, g)],
                                     dsts[nxt].at[pl.ds(0, g)], isems[nxt])
                pltpu.async_copy(x_hbm.at[sv.at[0]], rows_a, sem_a)

                def step(i, carry, sv=sv, dv=dv, g=g):
                    j0 = 2 * i
                    j1 = j0 + 1
                    pltpu.make_async_copy(
                        x_hbm.at[sv.at[0]], rows_a, sem_a).wait()
                    pltpu.async_copy(x_hbm.at[sv.at[j1]], rows_b, sem_b)
                    pltpu.sync_copy(rows_a, acc.at[dv.at[j0]], add=True)
                    pltpu.make_async_copy(
                        x_hbm.at[sv.at[0]], rows_b, sem_b).wait()

                    @pl.when(j1 + 1 < g)
                    def _():
                        pltpu.async_copy(
                            x_hbm.at[sv.at[j1 + 1]], rows_a, sem_a)

                    pltpu.sync_copy(rows_b, acc.at[dv.at[j1]], add=True)
                    return carry

                lax.fori_loop(0, g // 2, step, 0)

        @pl.when(cid == 0)
        def _():
            pipeline(sid * k0, g0, s0)

        @pl.when(cid == 1)
        def _():
            pipeline(NS * k0 + sid * k1, g1, s1)

        plsc.subcore_barrier()
        pltpu.sync_copy(acc.at[pl.ds(base, rows_pt)],
                        out_hbm.at[cid, pl.ds(base, rows_pt)])

    return kern(x, src_t, dst_t)


def _gru_body(parts_ref, x_ref, wih_ref, whh_ref, bih_ref, bhh_ref,
              gam_ref, bet_ref, o_ref):
    msg = parts_ref[0] + parts_ref[1]
    xb = x_ref[...]
    dn = (((1,), (1,)), ((), ()))
    gi = lax.dot_general(msg, wih_ref[...], dn,
                         preferred_element_type=jnp.float32) + bih_ref[...]
    gh = lax.dot_general(xb, whh_ref[...], dn,
                         preferred_element_type=jnp.float32) + bhh_ref[...]
    r = jax.nn.sigmoid(gi[:, :DIM] + gh[:, :DIM])
    z = jax.nn.sigmoid(gi[:, DIM:2 * DIM] + gh[:, DIM:2 * DIM])
    n = jnp.tanh(gi[:, 2 * DIM:] + r * gh[:, 2 * DIM:])
    h = (1.0 - z) * n + z * xb
    mu = jnp.mean(h, axis=1, keepdims=True)
    d = h - mu
    var = jnp.mean(d * d, axis=1, keepdims=True)
    o_ref[...] = d * lax.rsqrt(var + 1e-5) * gam_ref[...] + bet_ref[...]


def _gru_tc(parts, x, w_ih, w_hh, b_ih, b_hh, gamma, beta, blk):
    n = x.shape[0]
    grid = (n // blk,)
    return pl.pallas_call(
        _gru_body,
        grid=grid,
        in_specs=[
            pl.BlockSpec((NC, blk, DIM), lambda i: (0, i, 0)),
            pl.BlockSpec((blk, DIM), lambda i: (i, 0)),
            pl.BlockSpec((3 * DIM, DIM), lambda i: (0, 0)),
            pl.BlockSpec((3 * DIM, DIM), lambda i: (0, 0)),
            pl.BlockSpec((1, 3 * DIM), lambda i: (0, 0)),
            pl.BlockSpec((1, 3 * DIM), lambda i: (0, 0)),
            pl.BlockSpec((1, DIM), lambda i: (0, 0)),
            pl.BlockSpec((1, DIM), lambda i: (0, 0)),
        ],
        out_specs=pl.BlockSpec((blk, DIM), lambda i: (i, 0)),
        out_shape=jax.ShapeDtypeStruct((n, DIM), jnp.float32),
    )(parts, x, w_ih, w_hh, b_ih, b_hh, gamma, beta)


@jax.jit
def kernel(x, edge_index, W_ih, W_hh, b_ih, b_hh, ln_gamma, ln_beta):
    n = x.shape[0]
    e = edge_index.shape[1]

    per_u = CHUNK * NS * (K0 + K1)
    u = -(-e // per_u)
    e_pad = u * per_u
    src = jnp.concatenate(
        [edge_index[0], jnp.zeros((e_pad - e,), jnp.int32)]).reshape(-1, CHUNK)
    dst = jnp.concatenate(
        [edge_index[1], jnp.full((e_pad - e,), n, jnp.int32)]).reshape(-1, CHUNK)

    n_acc = -(-(n + 1) // (NS * 8)) * (NS * 8)

    parts = _sc_segment_sum(x, src, dst, n_acc, u)
    parts = lax.slice(parts, (0, 0, 0), (NC, n, DIM))

    return _gru_tc(parts, x, W_ih, W_hh, b_ih.reshape(1, -1),
                   b_hh.reshape(1, -1), ln_gamma.reshape(1, -1),
                   ln_beta.reshape(1, -1), blk=2000)
